# baseline (speedup 1.0000x reference)
import functools

import jax
import jax.numpy as jnp
from jax import lax
from jax.experimental import pallas as pl
from jax.experimental.pallas import tpu as pltpu
from jax.experimental.pallas import tpu_sc as plsc

NC = 2
NS = 16
NW = NC * NS


def _mesh():
    return plsc.VectorSubcoreMesh(
        core_axis_name="c", subcore_axis_name="s", num_cores=NC, num_subcores=NS
    )


def _make_sc_gather(npad, dtbl):
    rpw = npad // NW
    cg = 80
    ng = rpw // cg

    @functools.partial(
        pl.kernel,
        out_type=jax.ShapeDtypeStruct((npad, dtbl), jnp.float32),
        mesh=_mesh(),
        scratch_types=[
            pltpu.VMEM((1, cg), jnp.int32),
            pltpu.VMEM((cg, dtbl), jnp.float32),
            pltpu.SemaphoreType.DMA,
        ],
    )
    def k(table_hbm, idx_hbm, out_hbm, idx_v, rows_v, sem):
        wid = lax.axis_index("c") * NS + lax.axis_index("s")
        base = wid * rpw

        def body(g, _):
            b = base + g * cg
            pltpu.sync_copy(idx_hbm.at[pl.ds(b, cg)], idx_v.at[0])
            pltpu.async_copy(table_hbm.at[idx_v.at[0]], rows_v, sem).wait()
            pltpu.sync_copy(rows_v, out_hbm.at[pl.ds(b, cg)])
            return 0

        lax.fori_loop(0, ng, body, 0)

    return k


def _make_sc_agg(n, e, d):
    epw = e // NW
    c = 80
    nmain = epw // c
    tail = epw - nmain * c
    zr = n // NS

    assert tail == 0, "edge count must split into whole chunks per worker"
    ND = 3
    NI = 6
    scratch = [
        pltpu.VMEM((NI, c), jnp.int32),
        pltpu.VMEM((NI, c), jnp.int32),
    ] + [pltpu.VMEM((c, d), jnp.float32)] * ND \
      + [pltpu.VMEM((c // 2, d), jnp.int32)] * ND + [
        pltpu.VMEM_SHARED((n, d), jnp.float32),
    ] + [pltpu.SemaphoreType.DMA] * (3 * ND + NI)

    @functools.partial(
        pl.kernel,
        out_type=jax.ShapeDtypeStruct((NC, n, d), jnp.float32),
        mesh=_mesh(),
        scratch_types=scratch,
    )
    def k(h_hbm, src_hbm, dst_hbm, emb_hbm, zer_hbm, out_hbm,
          src_i, dst_i, *bufs_and_sems):
        rows = bufs_and_sems[:ND]
        embs = bufs_and_sems[ND:2 * ND]
        acc = bufs_and_sems[2 * ND]
        gsems = bufs_and_sems[2 * ND + 1:2 * ND + 1 + ND]
        esems = bufs_and_sems[2 * ND + 1 + ND:2 * ND + 1 + 2 * ND]
        ssems = bufs_and_sems[2 * ND + 1 + 2 * ND:2 * ND + 1 + 3 * ND]
        ipsems = bufs_and_sems[2 * ND + 1 + 3 * ND:]
        cid = lax.axis_index("c")
        sid = lax.axis_index("s")
        wid = cid * NS + sid
        pltpu.sync_copy(zer_hbm, acc.at[pl.ds(sid * zr, zr)])
        plsc.subcore_barrier()

        ebase = wid * epw

        def prefetch(g, si):
            base = ebase + g * c
            pltpu.async_copy(src_hbm.at[pl.ds(base, c)], src_i.at[si],
                             ipsems[si])
            pltpu.async_copy(dst_hbm.at[pl.ds(base, c)], dst_i.at[si],
                             ipsems[si])

        def drain_scatter(si, bd):
            pltpu.make_async_copy(rows[bd], acc.at[dst_i.at[si]],
                                  ssems[bd]).wait()

        def issue(g, si, bd):
            @pl.when(g >= ND)
            def _():
                drain_scatter((si + NI - ND) % NI, bd)

            base = ebase + g * c
            pltpu.make_async_copy(src_hbm.at[pl.ds(base, c)], src_i.at[si],
                                  ipsems[si]).wait()
            pltpu.make_async_copy(dst_hbm.at[pl.ds(base, c)], dst_i.at[si],
                                  ipsems[si]).wait()
            pltpu.async_copy(h_hbm.at[src_i.at[si]], rows[bd], gsems[bd])
            pltpu.async_copy(
                emb_hbm.at[pl.ds(pl.multiple_of(base // 2, 8), c // 2)],
                embs[bd], esems[bd])

        def consume(g, si, bd):
            pltpu.make_async_copy(h_hbm.at[src_i.at[si]], rows[bd],
                                  gsems[bd]).wait()
            base = ebase + g * c
            pltpu.make_async_copy(
                emb_hbm.at[pl.ds(pl.multiple_of(base // 2, 8), c // 2)],
                embs[bd], esems[bd]).wait()
            rv, ev = rows[bd], embs[bd]
            himask = jnp.full((16,), -65536, jnp.int32)

            @plsc.parallel_loop(0, c, 1, unroll=4)
            def row(r):
                for kk in range(d // 32):
                    vi = ev[r >> 1, pl.ds((r & 1) * (d // 2) + kk * 16, 16)]
                    lo = lax.bitcast_convert_type(vi << 16, jnp.float32)
                    hi = lax.bitcast_convert_type(vi & himask, jnp.float32)
                    sl0 = pl.ds(kk * 32, 16)
                    sl1 = pl.ds(kk * 32 + 16, 16)
                    rv[r, sl0] = jnp.maximum(rv[r, sl0] + lo, 0.0) + 1e-7
                    rv[r, sl1] = jnp.maximum(rv[r, sl1] + hi, 0.0) + 1e-7

            pltpu.async_copy(rv, acc.at[dst_i.at[si]], ssems[bd], add=True)

        for gp in range(ND):
            prefetch(gp, gp)
        for gp in range(ND - 1):
            issue(gp, gp, gp)

        def outer(k6, _):
            for b in range(NI):
                g = k6 * NI + b

                @pl.when(g + ND < nmain)
                def _():
                    prefetch(g + ND, (b + ND) % NI)

                @pl.when(g + ND - 1 < nmain)
                def _():
                    issue(g + ND - 1, (b + ND - 1) % NI, (b + ND - 1) % ND)

                @pl.when(g < nmain)
                def _():
                    consume(g, b % NI, b % ND)
            return 0

        lax.fori_loop(0, (nmain + NI - 1) // NI, outer, 0)
        for gl in range(nmain - ND, nmain):
            drain_scatter(gl % NI, gl % ND)

        plsc.subcore_barrier()

        @pl.when(sid == 0)
        def _():
            pltpu.sync_copy(acc, out_hbm.at[cid])

    return k


def _gate(a, w1, b1, w2row, b2):
    t = jnp.dot(a, w1, preferred_element_type=jnp.float32) + b1
    t = jnp.maximum(t, 0.2 * t)
    s = jnp.sum(t * w2row, axis=-1, keepdims=True) + b2
    return 1.0 / (1.0 + jnp.exp(-s))


def _layer_norm(h, g, b):
    mu = jnp.mean(h, axis=-1, keepdims=True)
    dl = h - mu
    var = jnp.mean(dl * dl, axis=-1, keepdims=True)
    return dl * lax.rsqrt(var + 1e-5) * g + b


def _prologue_body(x_ref, nf1e_ref, wohe, bohe, wnfe2, bnfe, w1, b1, w2, b2,
                   h_ref, cb_ref):
    x = x_ref[...]
    nf2 = jnp.dot(x, wohe[...], preferred_element_type=jnp.float32) + bohe[...]
    h = (nf1e_ref[...]
         + jnp.dot(nf2, wnfe2[...], preferred_element_type=jnp.float32)
         + bnfe[...])
    nw = _gate(h, w1[...], b1[...], w2[...], b2[...])
    h1 = h * nw
    h_ref[...] = h1
    cb_ref[...] = h1 * nw


def _edge_emb_body(ea_ref, we, be, out_ref):
    out_ref[...] = (jnp.dot(ea_ref[...], we[...],
                            preferred_element_type=jnp.float32) + be[...])


def _round_bf16_bits(x):
    return lax.bitcast_convert_type(x, jnp.int32) + 0x8000


def _pack_pair_words(y, d):
    words = []
    for g in range(d // 32):
        lo = _round_bf16_bits(y[:, g * 32:g * 32 + 16])
        hi = _round_bf16_bits(y[:, g * 32 + 16:g * 32 + 32])
        words.append(((lo >> 16) & 0xFFFF) | (hi & ~0xFFFF))
    return jnp.concatenate(words, axis=1)


def _edge_emb_packed_body(ea_ref, we, be, out_ref):
    y = (jnp.dot(ea_ref[...], we[...], preferred_element_type=jnp.float32)
         + be[...])
    eb, d = y.shape
    y3 = y.reshape(eb // 2, 2, d)
    out_ref[...] = jnp.concatenate(
        [_pack_pair_words(y3[:, 0, :], d), _pack_pair_words(y3[:, 1, :], d)],
        axis=1)


def _layer_core(h_ref, p_ref, cb_ref, wg, bg, lng, lnb, w1, b1, w2, b2):
    h = h_ref[...]
    s = h + p_ref[0] + p_ref[1]
    h1 = jnp.dot(s, wg[...], preferred_element_type=jnp.float32) + bg[...]
    hr = jnp.maximum(_layer_norm(h1, lng[...], lnb[...]), 0.0)
    cb = cb_ref[...]
    nw = _gate(hr + cb, w1[...], b1[...], w2[...], b2[...])
    hf = hr * nw
    return hf + cb * (1.0 - nw), cb + hf


def _layer_body(h_ref, p_ref, cb_ref, wg, bg, lng, lnb, w1, b1, w2, b2,
                ho_ref, cbo_ref):
    hn, cbn = _layer_core(h_ref, p_ref, cb_ref, wg, bg, lng, lnb, w1, b1, w2, b2)
    ho_ref[...] = hn
    cbo_ref[...] = cbn


def _layer_final_body(h_ref, p_ref, cb_ref, wg, bg, lng, lnb, w1, b1, w2, b2,
                      wp, bp, out_ref):
    _, cbn = _layer_core(h_ref, p_ref, cb_ref, wg, bg, lng, lnb, w1, b1, w2, b2)
    out_ref[...] = jnp.dot(cbn, wp[...], preferred_element_type=jnp.float32) + bp[...]


def _full(shape):
    nd = len(shape)
    return pl.BlockSpec(shape, lambda i: (0,) * nd)


def kernel(x, node_index, edge_index, edge_attr, node_features, W_ohe, b_ohe,
           W_nfe, b_nfe, W_edge, b_edge, Wg, bg, ln_g, ln_b, Wl1, bl1, Wl2,
           bl2, Wp, bp):
    n = x.shape[0]
    e = edge_index.shape[1]
    d = W_nfe.shape[1]
    hid = Wl1.shape[2]
    nlay = Wg.shape[0]
    t = Wp.shape[1]
    dtbl = node_features.shape[1]

    f32 = jnp.float32

    tbl = node_features.shape[0]
    tbl_b = 2000
    tbl2 = pl.pallas_call(
        _edge_emb_body,
        grid=(tbl // tbl_b,),
        in_specs=[
            pl.BlockSpec((tbl_b, dtbl), lambda i: (i, 0)),
            _full((dtbl, d)), _full((1, d)),
        ],
        out_specs=pl.BlockSpec((tbl_b, d), lambda i: (i, 0)),
        out_shape=jax.ShapeDtypeStruct((tbl, d), f32),
    )(node_features, W_nfe[:dtbl], jnp.zeros((1, d), f32))

    npad = ((n + 8 * NW - 1) // (8 * NW)) * (8 * NW)
    idx_pad = jnp.concatenate(
        [node_index, jnp.zeros((npad - n,), jnp.int32)]) if npad != n else node_index
    nf1e = _make_sc_gather(npad, d)(tbl2, idx_pad)[:n]

    rb = 400
    grid_n = n // rb
    b_ohe2 = b_ohe.reshape(1, -1)
    b_nfe2 = b_nfe.reshape(1, -1)
    h0, cb0 = pl.pallas_call(
        _prologue_body,
        grid=(grid_n,),
        in_specs=[
            pl.BlockSpec((rb, x.shape[1]), lambda i: (i, 0)),
            pl.BlockSpec((rb, d), lambda i: (i, 0)),
            _full(W_ohe.shape), _full(b_ohe2.shape),
            _full((x.shape[1], d)), _full(b_nfe2.shape),
            _full((d, hid)), _full((1, hid)), _full((1, hid)), _full((1, 1)),
        ],
        out_specs=[
            pl.BlockSpec((rb, d), lambda i: (i, 0)),
            pl.BlockSpec((rb, d), lambda i: (i, 0)),
        ],
        out_shape=[
            jax.ShapeDtypeStruct((n, d), f32),
            jax.ShapeDtypeStruct((n, d), f32),
        ],
    )(x, nf1e, W_ohe, b_ohe2, W_nfe[dtbl:], b_nfe2,
      Wl1[0], bl1[0].reshape(1, hid), Wl2[0, :, 0].reshape(1, hid),
      bl2[0].reshape(1, 1))

    eb = 2000
    embi = pl.pallas_call(
        _edge_emb_packed_body,
        grid=(e // eb,),
        in_specs=[
            pl.BlockSpec((eb, edge_attr.shape[1]), lambda i: (i, 0)),
            _full(W_edge.shape), _full((1, d)),
        ],
        out_specs=pl.BlockSpec((eb // 2, d), lambda i: (i, 0)),
        out_shape=jax.ShapeDtypeStruct((e // 2, d), jnp.int32),
    )(edge_attr, W_edge, b_edge.reshape(1, d))

    src = edge_index[0]
    dst = edge_index[1]
    zer = jnp.zeros((n // NS, d), f32)
    sc_agg = _make_sc_agg(n, e, d)

    tp = ((t + 127) // 128) * 128
    wp_pad = jnp.zeros((d, tp), f32).at[:, :t].set(Wp)
    bp_pad = jnp.zeros((1, tp), f32).at[0, :t].set(bp)

    layer_specs = [
        pl.BlockSpec((rb, d), lambda i: (i, 0)),
        pl.BlockSpec((NC, rb, d), lambda i: (0, i, 0)),
        pl.BlockSpec((rb, d), lambda i: (i, 0)),
        _full((d, d)), _full((1, d)), _full((1, d)), _full((1, d)),
        _full((d, hid)), _full((1, hid)), _full((1, hid)), _full((1, 1)),
    ]

    h, cb = h0, cb0
    for l in range(nlay):
        part = sc_agg(h, src, dst, embi, zer)
        wargs = (Wg[l], bg[l].reshape(1, d), ln_g[l].reshape(1, d),
                 ln_b[l].reshape(1, d), Wl1[l + 1], bl1[l + 1].reshape(1, hid),
                 Wl2[l + 1, :, 0].reshape(1, hid), bl2[l + 1].reshape(1, 1))
        if l + 1 < nlay:
            h, cb = pl.pallas_call(
                _layer_body,
                grid=(grid_n,),
                in_specs=layer_specs,
                out_specs=[
                    pl.BlockSpec((rb, d), lambda i: (i, 0)),
                    pl.BlockSpec((rb, d), lambda i: (i, 0)),
                ],
                out_shape=[
                    jax.ShapeDtypeStruct((n, d), f32),
                    jax.ShapeDtypeStruct((n, d), f32),
                ],
            )(h, part, cb, *wargs)
        else:
            out = pl.pallas_call(
                _layer_final_body,
                grid=(grid_n,),
                in_specs=layer_specs + [_full((d, tp)), _full((1, tp))],
                out_specs=pl.BlockSpec((rb, tp), lambda i: (i, 0)),
                out_shape=jax.ShapeDtypeStruct((n, tp), f32),
            )(h, part, cb, *wargs, wp_pad, bp_pad)

    return out[:, :t]


# speedup vs baseline: 1.0921x; 1.0921x over previous
import functools

import jax
import jax.numpy as jnp
from jax import lax
from jax.experimental import pallas as pl
from jax.experimental.pallas import tpu as pltpu
from jax.experimental.pallas import tpu_sc as plsc

NC = 2
NS = 16
NW = NC * NS


def _mesh():
    return plsc.VectorSubcoreMesh(
        core_axis_name="c", subcore_axis_name="s", num_cores=NC, num_subcores=NS
    )


def _make_sc_gather(npad, dtbl):
    rpw = npad // NW
    cg = 80
    ng = rpw // cg

    @functools.partial(
        pl.kernel,
        out_type=jax.ShapeDtypeStruct((npad, dtbl), jnp.float32),
        mesh=_mesh(),
        scratch_types=[
            pltpu.VMEM((1, cg), jnp.int32),
            pltpu.VMEM((cg, dtbl), jnp.float32),
            pltpu.SemaphoreType.DMA,
        ],
    )
    def k(table_hbm, idx_hbm, out_hbm, idx_v, rows_v, sem):
        wid = lax.axis_index("c") * NS + lax.axis_index("s")
        base = wid * rpw

        def body(g, _):
            b = base + g * cg
            pltpu.sync_copy(idx_hbm.at[pl.ds(b, cg)], idx_v.at[0])
            pltpu.async_copy(table_hbm.at[idx_v.at[0]], rows_v, sem).wait()
            pltpu.sync_copy(rows_v, out_hbm.at[pl.ds(b, cg)])
            return 0

        lax.fori_loop(0, ng, body, 0)

    return k


def _make_sc_agg(n, e, d):
    epw = e // NW
    c = 80
    nmain = epw // c
    tail = epw - nmain * c
    zr = n // NS

    assert tail == 0, "edge count must split into whole chunks per worker"
    ND = 3
    NI = 6
    scratch = [
        pltpu.VMEM((NI, c), jnp.int32),
        pltpu.VMEM((NI, c), jnp.int32),
    ] + [pltpu.VMEM((c, d), jnp.float32)] * ND \
      + [pltpu.VMEM((c // 2, d), jnp.int32)] * ND + [
        pltpu.VMEM_SHARED((n, d), jnp.float32),
    ] + [pltpu.SemaphoreType.DMA] * (3 * ND + NI)

    @functools.partial(
        pl.kernel,
        out_type=jax.ShapeDtypeStruct((NC, n, d), jnp.float32),
        mesh=_mesh(),
        scratch_types=scratch,
    )
    def k(h_hbm, src_hbm, dst_hbm, emb_hbm, zer_hbm, out_hbm,
          src_i, dst_i, *bufs_and_sems):
        rows = bufs_and_sems[:ND]
        embs = bufs_and_sems[ND:2 * ND]
        acc = bufs_and_sems[2 * ND]
        gsems = bufs_and_sems[2 * ND + 1:2 * ND + 1 + ND]
        esems = bufs_and_sems[2 * ND + 1 + ND:2 * ND + 1 + 2 * ND]
        ssems = bufs_and_sems[2 * ND + 1 + 2 * ND:2 * ND + 1 + 3 * ND]
        ipsems = bufs_and_sems[2 * ND + 1 + 3 * ND:]
        cid = lax.axis_index("c")
        sid = lax.axis_index("s")
        wid = cid * NS + sid
        pltpu.sync_copy(zer_hbm, acc.at[pl.ds(sid * zr, zr)])
        plsc.subcore_barrier()

        ebase = wid * epw

        def prefetch(g, si):
            base = ebase + g * c
            pltpu.async_copy(src_hbm.at[pl.ds(base, c)], src_i.at[si],
                             ipsems[si])
            pltpu.async_copy(dst_hbm.at[pl.ds(base, c)], dst_i.at[si],
                             ipsems[si])

        def drain_scatter(si, bd):
            pltpu.make_async_copy(rows[bd], acc.at[dst_i.at[si]],
                                  ssems[bd]).wait()

        def issue(g, si, bd):
            @pl.when(g >= ND)
            def _():
                drain_scatter((si + NI - ND) % NI, bd)

            base = ebase + g * c
            pltpu.make_async_copy(src_hbm.at[pl.ds(base, c)], src_i.at[si],
                                  ipsems[si]).wait()
            pltpu.make_async_copy(dst_hbm.at[pl.ds(base, c)], dst_i.at[si],
                                  ipsems[si]).wait()
            pltpu.async_copy(h_hbm.at[src_i.at[si]], rows[bd], gsems[bd])
            pltpu.async_copy(
                emb_hbm.at[pl.ds(pl.multiple_of(base // 2, 8), c // 2)],
                embs[bd], esems[bd])

        def consume(g, si, bd):
            pltpu.make_async_copy(h_hbm.at[src_i.at[si]], rows[bd],
                                  gsems[bd]).wait()
            base = ebase + g * c
            pltpu.make_async_copy(
                emb_hbm.at[pl.ds(pl.multiple_of(base // 2, 8), c // 2)],
                embs[bd], esems[bd]).wait()
            rv, ev = rows[bd], embs[bd]
            himask = jnp.full((16,), -65536, jnp.int32)

            @plsc.parallel_loop(0, c, 1, unroll=4)
            def row(r):
                for kk in range(d // 32):
                    vi = ev[r >> 1, pl.ds((r & 1) * (d // 2) + kk * 16, 16)]
                    lo = lax.bitcast_convert_type(vi << 16, jnp.float32)
                    hi = lax.bitcast_convert_type(vi & himask, jnp.float32)
                    sl0 = pl.ds(kk * 32, 16)
                    sl1 = pl.ds(kk * 32 + 16, 16)
                    rv[r, sl0] = jnp.maximum(rv[r, sl0] + lo, 0.0) + 1e-7
                    rv[r, sl1] = jnp.maximum(rv[r, sl1] + hi, 0.0) + 1e-7

            pltpu.async_copy(rv, acc.at[dst_i.at[si]], ssems[bd], add=True)

        for gp in range(ND):
            prefetch(gp, gp)
        for gp in range(ND - 1):
            issue(gp, gp, gp)

        def outer(k6, _):
            for b in range(NI):
                g = k6 * NI + b

                @pl.when(g + ND < nmain)
                def _():
                    prefetch(g + ND, (b + ND) % NI)

                @pl.when(g + ND - 1 < nmain)
                def _():
                    issue(g + ND - 1, (b + ND - 1) % NI, (b + ND - 1) % ND)

                @pl.when(g < nmain)
                def _():
                    consume(g, b % NI, b % ND)
            return 0

        lax.fori_loop(0, (nmain + NI - 1) // NI, outer, 0)
        for gl in range(nmain - ND, nmain):
            drain_scatter(gl % NI, gl % ND)

        plsc.subcore_barrier()

        @pl.when(sid == 0)
        def _():
            pltpu.sync_copy(acc, out_hbm.at[cid])

    return k


def _gate(a, w1, b1, w2row, b2):
    t = jnp.dot(a, w1, preferred_element_type=jnp.float32) + b1
    t = jnp.maximum(t, 0.2 * t)
    s = jnp.sum(t * w2row, axis=-1, keepdims=True) + b2
    return 1.0 / (1.0 + jnp.exp(-s))


def _layer_norm(h, g, b):
    mu = jnp.mean(h, axis=-1, keepdims=True)
    dl = h - mu
    var = jnp.mean(dl * dl, axis=-1, keepdims=True)
    return dl * lax.rsqrt(var + 1e-5) * g + b


def _prologue_body(x_ref, nf1e_ref, wohe, bohe, wnfe2, bnfe, w1, b1, w2, b2,
                   h_ref, cb_ref):
    x = x_ref[...]
    nf2 = jnp.dot(x, wohe[...], preferred_element_type=jnp.float32) + bohe[...]
    h = (nf1e_ref[...]
         + jnp.dot(nf2, wnfe2[...], preferred_element_type=jnp.float32)
         + bnfe[...])
    nw = _gate(h, w1[...], b1[...], w2[...], b2[...])
    h1 = h * nw
    h_ref[...] = h1
    cb_ref[...] = h1 * nw


def _edge_emb_body(ea_ref, we, be, out_ref):
    out_ref[...] = (jnp.dot(ea_ref[...], we[...],
                            preferred_element_type=jnp.float32) + be[...])


def _round_bf16_bits(x):
    return lax.bitcast_convert_type(x, jnp.int32) + 0x8000


def _pack_pair_words(y, d):
    words = []
    for g in range(d // 32):
        lo = _round_bf16_bits(y[:, g * 32:g * 32 + 16])
        hi = _round_bf16_bits(y[:, g * 32 + 16:g * 32 + 32])
        words.append(((lo >> 16) & 0xFFFF) | (hi & ~0xFFFF))
    return jnp.concatenate(words, axis=1)


def _edge_emb_packed_body(ea2_ref, we, be, out_ref):
    ea2 = ea2_ref[...]
    k = ea2.shape[1] // 2
    d = we.shape[1]
    ye = jnp.dot(ea2[:, :k], we[...], preferred_element_type=jnp.float32) + be[...]
    yo = jnp.dot(ea2[:, k:], we[...], preferred_element_type=jnp.float32) + be[...]
    out_ref[...] = jnp.concatenate(
        [_pack_pair_words(ye, d), _pack_pair_words(yo, d)], axis=1)


def _layer_core(h_ref, p_ref, cb_ref, wg, bg, lng, lnb, w1, b1, w2, b2):
    h = h_ref[...]
    s = h + p_ref[0] + p_ref[1]
    h1 = jnp.dot(s, wg[...], preferred_element_type=jnp.float32) + bg[...]
    hr = jnp.maximum(_layer_norm(h1, lng[...], lnb[...]), 0.0)
    cb = cb_ref[...]
    nw = _gate(hr + cb, w1[...], b1[...], w2[...], b2[...])
    hf = hr * nw
    return hf + cb * (1.0 - nw), cb + hf


def _layer_body(h_ref, p_ref, cb_ref, wg, bg, lng, lnb, w1, b1, w2, b2,
                ho_ref, cbo_ref):
    hn, cbn = _layer_core(h_ref, p_ref, cb_ref, wg, bg, lng, lnb, w1, b1, w2, b2)
    ho_ref[...] = hn
    cbo_ref[...] = cbn


def _layer_final_body(h_ref, p_ref, cb_ref, wg, bg, lng, lnb, w1, b1, w2, b2,
                      wp, bp, out_ref):
    _, cbn = _layer_core(h_ref, p_ref, cb_ref, wg, bg, lng, lnb, w1, b1, w2, b2)
    out_ref[...] = jnp.dot(cbn, wp[...], preferred_element_type=jnp.float32) + bp[...]


def _full(shape):
    nd = len(shape)
    return pl.BlockSpec(shape, lambda i: (0,) * nd)


def kernel(x, node_index, edge_index, edge_attr, node_features, W_ohe, b_ohe,
           W_nfe, b_nfe, W_edge, b_edge, Wg, bg, ln_g, ln_b, Wl1, bl1, Wl2,
           bl2, Wp, bp):
    n = x.shape[0]
    e = edge_index.shape[1]
    d = W_nfe.shape[1]
    hid = Wl1.shape[2]
    nlay = Wg.shape[0]
    t = Wp.shape[1]
    dtbl = node_features.shape[1]

    f32 = jnp.float32

    tbl = node_features.shape[0]
    tbl_b = 2000
    tbl2 = pl.pallas_call(
        _edge_emb_body,
        grid=(tbl // tbl_b,),
        in_specs=[
            pl.BlockSpec((tbl_b, dtbl), lambda i: (i, 0)),
            _full((dtbl, d)), _full((1, d)),
        ],
        out_specs=pl.BlockSpec((tbl_b, d), lambda i: (i, 0)),
        out_shape=jax.ShapeDtypeStruct((tbl, d), f32),
    )(node_features, W_nfe[:dtbl], jnp.zeros((1, d), f32))

    npad = ((n + 8 * NW - 1) // (8 * NW)) * (8 * NW)
    idx_pad = jnp.concatenate(
        [node_index, jnp.zeros((npad - n,), jnp.int32)]) if npad != n else node_index
    nf1e = _make_sc_gather(npad, d)(tbl2, idx_pad)[:n]

    rb = 400
    grid_n = n // rb
    b_ohe2 = b_ohe.reshape(1, -1)
    b_nfe2 = b_nfe.reshape(1, -1)
    h0, cb0 = pl.pallas_call(
        _prologue_body,
        grid=(grid_n,),
        in_specs=[
            pl.BlockSpec((rb, x.shape[1]), lambda i: (i, 0)),
            pl.BlockSpec((rb, d), lambda i: (i, 0)),
            _full(W_ohe.shape), _full(b_ohe2.shape),
            _full((x.shape[1], d)), _full(b_nfe2.shape),
            _full((d, hid)), _full((1, hid)), _full((1, hid)), _full((1, 1)),
        ],
        out_specs=[
            pl.BlockSpec((rb, d), lambda i: (i, 0)),
            pl.BlockSpec((rb, d), lambda i: (i, 0)),
        ],
        out_shape=[
            jax.ShapeDtypeStruct((n, d), f32),
            jax.ShapeDtypeStruct((n, d), f32),
        ],
    )(x, nf1e, W_ohe, b_ohe2, W_nfe[dtbl:], b_nfe2,
      Wl1[0], bl1[0].reshape(1, hid), Wl2[0, :, 0].reshape(1, hid),
      bl2[0].reshape(1, 1))

    eb = 2000
    ka = edge_attr.shape[1]
    embi = pl.pallas_call(
        _edge_emb_packed_body,
        grid=(e // eb,),
        in_specs=[
            pl.BlockSpec((eb // 2, 2 * ka), lambda i: (i, 0)),
            _full(W_edge.shape), _full((1, d)),
        ],
        out_specs=pl.BlockSpec((eb // 2, d), lambda i: (i, 0)),
        out_shape=jax.ShapeDtypeStruct((e // 2, d), jnp.int32),
    )(edge_attr.reshape(e // 2, 2 * ka), W_edge, b_edge.reshape(1, d))

    src = edge_index[0]
    dst = edge_index[1]
    zer = jnp.zeros((n // NS, d), f32)
    sc_agg = _make_sc_agg(n, e, d)

    tp = ((t + 127) // 128) * 128
    wp_pad = jnp.zeros((d, tp), f32).at[:, :t].set(Wp)
    bp_pad = jnp.zeros((1, tp), f32).at[0, :t].set(bp)

    layer_specs = [
        pl.BlockSpec((rb, d), lambda i: (i, 0)),
        pl.BlockSpec((NC, rb, d), lambda i: (0, i, 0)),
        pl.BlockSpec((rb, d), lambda i: (i, 0)),
        _full((d, d)), _full((1, d)), _full((1, d)), _full((1, d)),
        _full((d, hid)), _full((1, hid)), _full((1, hid)), _full((1, 1)),
    ]

    h, cb = h0, cb0
    for l in range(nlay):
        part = sc_agg(h, src, dst, embi, zer)
        wargs = (Wg[l], bg[l].reshape(1, d), ln_g[l].reshape(1, d),
                 ln_b[l].reshape(1, d), Wl1[l + 1], bl1[l + 1].reshape(1, hid),
                 Wl2[l + 1, :, 0].reshape(1, hid), bl2[l + 1].reshape(1, 1))
        if l + 1 < nlay:
            h, cb = pl.pallas_call(
                _layer_body,
                grid=(grid_n,),
                in_specs=layer_specs,
                out_specs=[
                    pl.BlockSpec((rb, d), lambda i: (i, 0)),
                    pl.BlockSpec((rb, d), lambda i: (i, 0)),
                ],
                out_shape=[
                    jax.ShapeDtypeStruct((n, d), f32),
                    jax.ShapeDtypeStruct((n, d), f32),
                ],
            )(h, part, cb, *wargs)
        else:
            out = pl.pallas_call(
                _layer_final_body,
                grid=(grid_n,),
                in_specs=layer_specs + [_full((d, tp)), _full((1, tp))],
                out_specs=pl.BlockSpec((rb, tp), lambda i: (i, 0)),
                out_shape=jax.ShapeDtypeStruct((n, tp), f32),
            )(h, part, cb, *wargs, wp_pad, bp_pad)

    return out[:, :t]


# speedup vs baseline: 1.1236x; 1.0289x over previous
import functools

import jax
import jax.numpy as jnp
from jax import lax
from jax.experimental import pallas as pl
from jax.experimental.pallas import tpu as pltpu
from jax.experimental.pallas import tpu_sc as plsc

NC = 2
NS = 16
NW = NC * NS


def _mesh():
    return plsc.VectorSubcoreMesh(
        core_axis_name="c", subcore_axis_name="s", num_cores=NC, num_subcores=NS
    )


def _make_sc_gather(npad, dtbl):
    rpw = npad // NW
    cg = 80
    ng = rpw // cg

    @functools.partial(
        pl.kernel,
        out_type=jax.ShapeDtypeStruct((npad, dtbl), jnp.float32),
        mesh=_mesh(),
        scratch_types=[
            pltpu.VMEM((1, cg), jnp.int32),
            pltpu.VMEM((cg, dtbl), jnp.float32),
            pltpu.SemaphoreType.DMA,
        ],
    )
    def k(table_hbm, idx_hbm, out_hbm, idx_v, rows_v, sem):
        wid = lax.axis_index("c") * NS + lax.axis_index("s")
        base = wid * rpw

        def body(g, _):
            b = base + g * cg
            pltpu.sync_copy(idx_hbm.at[pl.ds(b, cg)], idx_v.at[0])
            pltpu.async_copy(table_hbm.at[idx_v.at[0]], rows_v, sem).wait()
            pltpu.sync_copy(rows_v, out_hbm.at[pl.ds(b, cg)])
            return 0

        lax.fori_loop(0, ng, body, 0)

    return k


def _make_sc_agg(n, e, d):
    epw = e // NW
    c = 80
    nmain = epw // c
    tail = epw - nmain * c
    zr = n // NS

    assert tail == 0, "edge count must split into whole chunks per worker"
    ND = 3
    NI = 6
    scratch = [
        pltpu.VMEM((NI, c), jnp.int32),
        pltpu.VMEM((NI, c), jnp.int32),
    ] + [pltpu.VMEM((c, d), jnp.float32)] * ND \
      + [pltpu.VMEM((c // 2, d), jnp.int32)] * ND + [
        pltpu.VMEM_SHARED((n, d), jnp.float32),
    ] + [pltpu.SemaphoreType.DMA] * (3 * ND + NI)

    @functools.partial(
        pl.kernel,
        out_type=jax.ShapeDtypeStruct((NC, n, d), jnp.float32),
        mesh=_mesh(),
        scratch_types=scratch,
    )
    def k(h_hbm, src_hbm, dst_hbm, emb_hbm, zer_hbm, out_hbm,
          src_i, dst_i, *bufs_and_sems):
        rows = bufs_and_sems[:ND]
        embs = bufs_and_sems[ND:2 * ND]
        acc = bufs_and_sems[2 * ND]
        gsems = bufs_and_sems[2 * ND + 1:2 * ND + 1 + ND]
        esems = bufs_and_sems[2 * ND + 1 + ND:2 * ND + 1 + 2 * ND]
        ssems = bufs_and_sems[2 * ND + 1 + 2 * ND:2 * ND + 1 + 3 * ND]
        ipsems = bufs_and_sems[2 * ND + 1 + 3 * ND:]
        cid = lax.axis_index("c")
        sid = lax.axis_index("s")
        wid = cid * NS + sid
        pltpu.sync_copy(zer_hbm, acc.at[pl.ds(sid * zr, zr)])
        plsc.subcore_barrier()

        ebase = wid * epw

        def prefetch(g, si):
            base = ebase + g * c
            pltpu.async_copy(src_hbm.at[pl.ds(base, c)], src_i.at[si],
                             ipsems[si])
            pltpu.async_copy(dst_hbm.at[pl.ds(base, c)], dst_i.at[si],
                             ipsems[si])

        def drain_scatter(si, bd):
            pltpu.make_async_copy(rows[bd], acc.at[dst_i.at[si]],
                                  ssems[bd]).wait()

        def issue(g, si, bd):
            @pl.when(g >= ND)
            def _():
                drain_scatter((si + NI - ND) % NI, bd)

            base = ebase + g * c
            pltpu.make_async_copy(src_hbm.at[pl.ds(base, c)], src_i.at[si],
                                  ipsems[si]).wait()
            pltpu.make_async_copy(dst_hbm.at[pl.ds(base, c)], dst_i.at[si],
                                  ipsems[si]).wait()
            pltpu.async_copy(h_hbm.at[src_i.at[si]], rows[bd], gsems[bd])
            pltpu.async_copy(
                emb_hbm.at[pl.ds(pl.multiple_of(base // 2, 8), c // 2)],
                embs[bd], esems[bd])

        def consume(g, si, bd):
            pltpu.make_async_copy(h_hbm.at[src_i.at[si]], rows[bd],
                                  gsems[bd]).wait()
            base = ebase + g * c
            pltpu.make_async_copy(
                emb_hbm.at[pl.ds(pl.multiple_of(base // 2, 8), c // 2)],
                embs[bd], esems[bd]).wait()
            rv, ev = rows[bd], embs[bd]
            himask = jnp.full((16,), -65536, jnp.int32)

            @plsc.parallel_loop(0, c, 1, unroll=4)
            def row(r):
                for kk in range(d // 32):
                    vi = ev[r >> 1, pl.ds((r & 1) * (d // 2) + kk * 16, 16)]
                    lo = lax.bitcast_convert_type(vi << 16, jnp.float32)
                    hi = lax.bitcast_convert_type(vi & himask, jnp.float32)
                    sl0 = pl.ds(kk * 32, 16)
                    sl1 = pl.ds(kk * 32 + 16, 16)
                    rv[r, sl0] = jnp.maximum(rv[r, sl0] + lo, 0.0) + 1e-7
                    rv[r, sl1] = jnp.maximum(rv[r, sl1] + hi, 0.0) + 1e-7

            pltpu.async_copy(rv, acc.at[dst_i.at[si]], ssems[bd], add=True)

        for gp in range(ND):
            prefetch(gp, gp)
        for gp in range(ND - 1):
            issue(gp, gp, gp)

        def outer(k6, _):
            for b in range(NI):
                g = k6 * NI + b

                @pl.when(g + ND < nmain)
                def _():
                    prefetch(g + ND, (b + ND) % NI)

                @pl.when(g + ND - 1 < nmain)
                def _():
                    issue(g + ND - 1, (b + ND - 1) % NI, (b + ND - 1) % ND)

                @pl.when(g < nmain)
                def _():
                    consume(g, b % NI, b % ND)
            return 0

        lax.fori_loop(0, (nmain + NI - 1) // NI, outer, 0)
        for gl in range(nmain - ND, nmain):
            drain_scatter(gl % NI, gl % ND)

        plsc.subcore_barrier()

        @pl.when(sid == 0)
        def _():
            pltpu.sync_copy(acc, out_hbm.at[cid])

    return k


def _gate(a, w1, b1, w2row, b2):
    t = jnp.dot(a, w1, preferred_element_type=jnp.float32) + b1
    t = jnp.maximum(t, 0.2 * t)
    s = jnp.sum(t * w2row, axis=-1, keepdims=True) + b2
    return 1.0 / (1.0 + jnp.exp(-s))


def _layer_norm(h, g, b):
    mu = jnp.mean(h, axis=-1, keepdims=True)
    dl = h - mu
    var = jnp.mean(dl * dl, axis=-1, keepdims=True)
    return dl * lax.rsqrt(var + 1e-5) * g + b


def _prologue_body(x_ref, nf1e_ref, wohe, bohe, wnfe2, bnfe, w1, b1, w2, b2,
                   h_ref, cb_ref):
    x = x_ref[...]
    nf2 = jnp.dot(x, wohe[...], preferred_element_type=jnp.float32) + bohe[...]
    h = (nf1e_ref[...]
         + jnp.dot(nf2, wnfe2[...], preferred_element_type=jnp.float32)
         + bnfe[...])
    nw = _gate(h, w1[...], b1[...], w2[...], b2[...])
    h1 = h * nw
    h_ref[...] = h1
    cb_ref[...] = h1 * nw


def _edge_emb_body(ea_ref, we, be, out_ref):
    out_ref[...] = (jnp.dot(ea_ref[...], we[...],
                            preferred_element_type=jnp.float32) + be[...])


def _round_bf16_bits(x):
    return lax.bitcast_convert_type(x, jnp.int32) + 0x8000


def _pack_pair_words(y, d):
    words = []
    for g in range(d // 32):
        lo = _round_bf16_bits(y[:, g * 32:g * 32 + 16])
        hi = _round_bf16_bits(y[:, g * 32 + 16:g * 32 + 32])
        words.append(((lo >> 16) & 0xFFFF) | (hi & ~0xFFFF))
    return jnp.concatenate(words, axis=1)


def _edge_emb_packed_body(ea2_ref, we, be, out_ref):
    ea2 = ea2_ref[...]
    k = ea2.shape[1] // 2
    d = we.shape[1]
    ye = jnp.dot(ea2[:, :k], we[...], preferred_element_type=jnp.float32) + be[...]
    yo = jnp.dot(ea2[:, k:], we[...], preferred_element_type=jnp.float32) + be[...]
    out_ref[...] = jnp.concatenate(
        [_pack_pair_words(ye, d), _pack_pair_words(yo, d)], axis=1)


def _layer_core(h_ref, p_ref, cb_ref, wg, bg, lng, lnb, w1, b1, w2, b2):
    h = h_ref[...]
    s = h + p_ref[0] + p_ref[1]
    h1 = jnp.dot(s, wg[...], preferred_element_type=jnp.float32) + bg[...]
    hr = jnp.maximum(_layer_norm(h1, lng[...], lnb[...]), 0.0)
    cb = cb_ref[...]
    nw = _gate(hr + cb, w1[...], b1[...], w2[...], b2[...])
    hf = hr * nw
    return hf + cb * (1.0 - nw), cb + hf


def _layer_body(h_ref, p_ref, cb_ref, wg, bg, lng, lnb, w1, b1, w2, b2,
                ho_ref, cbo_ref):
    hn, cbn = _layer_core(h_ref, p_ref, cb_ref, wg, bg, lng, lnb, w1, b1, w2, b2)
    ho_ref[...] = hn
    cbo_ref[...] = cbn


def _layer_final_body(h_ref, p_ref, cb_ref, wg, bg, lng, lnb, w1, b1, w2, b2,
                      wp, bp, out_ref):
    _, cbn = _layer_core(h_ref, p_ref, cb_ref, wg, bg, lng, lnb, w1, b1, w2, b2)
    out_ref[...] = jnp.dot(cbn, wp[...], preferred_element_type=jnp.float32) + bp[...]


def _full(shape):
    nd = len(shape)
    return pl.BlockSpec(shape, lambda i: (0,) * nd)


def kernel(x, node_index, edge_index, edge_attr, node_features, W_ohe, b_ohe,
           W_nfe, b_nfe, W_edge, b_edge, Wg, bg, ln_g, ln_b, Wl1, bl1, Wl2,
           bl2, Wp, bp):
    n = x.shape[0]
    e = edge_index.shape[1]
    d = W_nfe.shape[1]
    hid = Wl1.shape[2]
    nlay = Wg.shape[0]
    t = Wp.shape[1]
    dtbl = node_features.shape[1]

    f32 = jnp.float32

    tbl = node_features.shape[0]
    tbl_b = 2000
    tbl2 = pl.pallas_call(
        _edge_emb_body,
        grid=(tbl // tbl_b,),
        in_specs=[
            pl.BlockSpec((tbl_b, dtbl), lambda i: (i, 0)),
            _full((dtbl, d)), _full((1, d)),
        ],
        out_specs=pl.BlockSpec((tbl_b, d), lambda i: (i, 0)),
        out_shape=jax.ShapeDtypeStruct((tbl, d), f32),
    )(node_features, W_nfe[:dtbl], jnp.zeros((1, d), f32))

    npad = ((n + 8 * NW - 1) // (8 * NW)) * (8 * NW)
    idx_pad = jnp.concatenate(
        [node_index, jnp.zeros((npad - n,), jnp.int32)]) if npad != n else node_index
    nf1e = _make_sc_gather(npad, d)(tbl2, idx_pad)[:n]

    rb = 1000
    grid_n = n // rb
    b_ohe2 = b_ohe.reshape(1, -1)
    b_nfe2 = b_nfe.reshape(1, -1)
    h0, cb0 = pl.pallas_call(
        _prologue_body,
        grid=(grid_n,),
        in_specs=[
            pl.BlockSpec((rb, x.shape[1]), lambda i: (i, 0)),
            pl.BlockSpec((rb, d), lambda i: (i, 0)),
            _full(W_ohe.shape), _full(b_ohe2.shape),
            _full((x.shape[1], d)), _full(b_nfe2.shape),
            _full((d, hid)), _full((1, hid)), _full((1, hid)), _full((1, 1)),
        ],
        out_specs=[
            pl.BlockSpec((rb, d), lambda i: (i, 0)),
            pl.BlockSpec((rb, d), lambda i: (i, 0)),
        ],
        out_shape=[
            jax.ShapeDtypeStruct((n, d), f32),
            jax.ShapeDtypeStruct((n, d), f32),
        ],
    )(x, nf1e, W_ohe, b_ohe2, W_nfe[dtbl:], b_nfe2,
      Wl1[0], bl1[0].reshape(1, hid), Wl2[0, :, 0].reshape(1, hid),
      bl2[0].reshape(1, 1))

    eb = 2000
    ka = edge_attr.shape[1]
    embi = pl.pallas_call(
        _edge_emb_packed_body,
        grid=(e // eb,),
        in_specs=[
            pl.BlockSpec((eb // 2, 2 * ka), lambda i: (i, 0)),
            _full(W_edge.shape), _full((1, d)),
        ],
        out_specs=pl.BlockSpec((eb // 2, d), lambda i: (i, 0)),
        out_shape=jax.ShapeDtypeStruct((e // 2, d), jnp.int32),
    )(edge_attr.reshape(e // 2, 2 * ka), W_edge, b_edge.reshape(1, d))

    src = edge_index[0]
    dst = edge_index[1]
    zer = jnp.zeros((n // NS, d), f32)
    sc_agg = _make_sc_agg(n, e, d)

    tp = ((t + 127) // 128) * 128
    wp_pad = jnp.zeros((d, tp), f32).at[:, :t].set(Wp)
    bp_pad = jnp.zeros((1, tp), f32).at[0, :t].set(bp)

    layer_specs = [
        pl.BlockSpec((rb, d), lambda i: (i, 0)),
        pl.BlockSpec((NC, rb, d), lambda i: (0, i, 0)),
        pl.BlockSpec((rb, d), lambda i: (i, 0)),
        _full((d, d)), _full((1, d)), _full((1, d)), _full((1, d)),
        _full((d, hid)), _full((1, hid)), _full((1, hid)), _full((1, 1)),
    ]

    h, cb = h0, cb0
    for l in range(nlay):
        part = sc_agg(h, src, dst, embi, zer)
        wargs = (Wg[l], bg[l].reshape(1, d), ln_g[l].reshape(1, d),
                 ln_b[l].reshape(1, d), Wl1[l + 1], bl1[l + 1].reshape(1, hid),
                 Wl2[l + 1, :, 0].reshape(1, hid), bl2[l + 1].reshape(1, 1))
        if l + 1 < nlay:
            h, cb = pl.pallas_call(
                _layer_body,
                grid=(grid_n,),
                in_specs=layer_specs,
                out_specs=[
                    pl.BlockSpec((rb, d), lambda i: (i, 0)),
                    pl.BlockSpec((rb, d), lambda i: (i, 0)),
                ],
                out_shape=[
                    jax.ShapeDtypeStruct((n, d), f32),
                    jax.ShapeDtypeStruct((n, d), f32),
                ],
            )(h, part, cb, *wargs)
        else:
            out = pl.pallas_call(
                _layer_final_body,
                grid=(grid_n,),
                in_specs=layer_specs + [_full((d, tp)), _full((1, tp))],
                out_specs=pl.BlockSpec((rb, tp), lambda i: (i, 0)),
                out_shape=jax.ShapeDtypeStruct((n, tp), f32),
            )(h, part, cb, *wargs, wp_pad, bp_pad)

    return out[:, :t]
